# Initial kernel scaffold; baseline (speedup 1.0000x reference)
import jax
import jax.numpy as jnp
from jax.experimental import pallas as pl


def kernel(tracker, head_idx, seq_idx, compute_idx):
    raise NotImplementedError("write your pallas kernel here")


# speedup vs baseline: 7.0806x; 7.0806x over previous
import jax
import jax.numpy as jnp
from jax import lax
from jax.experimental import pallas as pl
from jax.experimental.pallas import tpu as pltpu
from jax.experimental.pallas import tpu_sc as plsc

_L = 16


def _tracker_update_body(trk_hbm, head_hbm, cmp_hbm, out_hbm,
                         head_s, cmp_v, row_v):
    num_sel = head_hbm.shape[0]
    width = cmp_hbm.shape[1]
    wid = lax.axis_index("s") * 2 + lax.axis_index("c")

    pltpu.sync_copy(head_hbm, head_s)
    neg1 = jnp.full((_L,), -1, jnp.int32)

    for rr in range(2):
        r = wid * 2 + rr

        sel = jnp.bool_(False)
        j = jnp.int32(0)
        for c in range(num_sel // _L):
            hv = head_s[pl.ds(c * _L, _L)]
            for i in range(_L):
                hit = hv[i] == r
                sel = sel | hit
                j = jnp.where(hit, jnp.int32(c * _L + i), j)

        pltpu.sync_copy(trk_hbm.at[2 * r + 1], row_v)
        pltpu.sync_copy(row_v, out_hbm.at[2 * r + 1])

        @pl.when(sel)
        def _():
            pltpu.sync_copy(cmp_hbm.at[j], cmp_v)
            pltpu.sync_copy(trk_hbm.at[2 * r], row_v)

            def mask_body(k, carry):
                base = k * _L
                cv = cmp_v[pl.ds(base, _L)]
                tv = row_v[pl.ds(base, _L)]
                row_v[pl.ds(base, _L)] = jnp.where(cv != -1, neg1, tv)
                return carry

            lax.fori_loop(0, width // _L, mask_body, 0)
            pltpu.sync_copy(row_v, out_hbm.at[2 * r])

        @pl.when(jnp.logical_not(sel))
        def _():
            pltpu.sync_copy(trk_hbm.at[2 * r], row_v)
            pltpu.sync_copy(row_v, out_hbm.at[2 * r])


def kernel(tracker, head_idx, seq_idx, compute_idx):
    num_heads, row_len = tracker.shape
    num_sel, width = compute_idx.shape
    del seq_idx
    trk2 = tracker.reshape(2 * num_heads, width)

    kern = pl.kernel(
        _tracker_update_body,
        out_type=jax.ShapeDtypeStruct((2 * num_heads, width), jnp.int32),
        mesh=plsc.VectorSubcoreMesh(core_axis_name="c", subcore_axis_name="s"),
        scratch_types=[
            pltpu.VMEM((num_sel,), jnp.int32),
            pltpu.VMEM((width,), jnp.int32),
            pltpu.VMEM((width,), jnp.int32),
        ],
    )
    out2 = kern(trk2, head_idx, compute_idx)
    return out2.reshape(num_heads, row_len)
